# baseline (speedup 1.0000x reference)
import functools
import math

import jax
import jax.numpy as jnp
import numpy as np
from jax import lax
from jax.experimental import pallas as pl
from jax.experimental.pallas import tpu as pltpu
from jax.experimental.pallas import tpu_sc as plsc

IMAGE_SIZE = 256
PAD = 5
PADDED = IMAGE_SIZE + 2 * PAD
_HWP = PADDED * PADDED
N_POINTS = 4 * 16384
N_SUBCORES = 32
PTS_PER_SUB = N_POINTS // N_SUBCORES
KR = 5
NSLOT = 112

_EYES = np.array(
    [[-1, -1, -1], [-1, -1, 1], [-1, 1, -1], [-1, 1, 1],
     [1, -1, -1], [1, -1, 1], [1, 1, -1], [1, 1, 1]], dtype=np.float32)

_C1 = float((np.pi / 10.0) ** 2)
_POLY = [(-1.0) ** k * _C1 ** k / float(math.factorial(2 * k))
         for k in range(5)]


def _normalize(x):
    n = jnp.linalg.norm(x, axis=1, keepdims=True)
    return x / jnp.maximum(n, 1e-6)


def _look_at(eyes, centers, ups):
    B = eyes.shape[0]
    zaxis = _normalize(eyes - centers)
    xaxis = _normalize(jnp.cross(ups, zaxis))
    yaxis = jnp.cross(zaxis, xaxis)
    z = jnp.zeros([B], dtype=eyes.dtype)
    o = jnp.ones([B], dtype=eyes.dtype)
    translation = jnp.stack(
        [o, z, z, -eyes[:, 0], z, o, z, -eyes[:, 1], z, z, o, -eyes[:, 2],
         z, z, z, o], -1).reshape(-1, 4, 4)
    orientation = jnp.stack(
        [xaxis[:, 0], xaxis[:, 1], xaxis[:, 2], z,
         yaxis[:, 0], yaxis[:, 1], yaxis[:, 2], z,
         zaxis[:, 0], zaxis[:, 1], zaxis[:, 2], z, z, z, z, o],
        -1).reshape(-1, 4, 4)
    return orientation @ translation


def _orthorgonal(scalex, scaley, z_near, z_far):
    z = jnp.zeros_like(z_near)
    o = jnp.ones_like(z_near)
    k1 = -2.0 / (z_far - z_near)
    k2 = (z_far + z_near) / (z_far - z_near)
    return jnp.stack(
        [scalex, z, z, z, z, scaley, z, z, z, z, k1, k2, z, z, z, o],
        -1).reshape(-1, 4, 4)


def _pre_matrices():
    proj = _orthorgonal(jnp.array([1.5], dtype=jnp.float32),
                        jnp.array([1.5], dtype=jnp.float32),
                        jnp.array([0.1], dtype=jnp.float32),
                        jnp.array([10.0], dtype=jnp.float32))
    mats = []
    for i in range(_EYES.shape[0]):
        vm = _look_at(jnp.asarray(_EYES[i:i + 1]),
                      jnp.zeros((1, 3), dtype=jnp.float32),
                      jnp.array([[0.0, 0.0, 1.0]], dtype=jnp.float32))
        mats.append(proj @ vm)
    return jnp.concatenate(mats, axis=0)


def _offset_tables():
    live, dead = [], []
    for di in range(-KR, KR + 1):
        for dj in range(-KR, KR + 1):
            m = max(abs(di) - 0.5, 0.0) ** 2 + max(abs(dj) - 0.5, 0.0) ** 2
            (live if m <= 25.0 else dead).append((di, dj))
    slots = live + dead[:NSLOT - len(live)]
    assert len(slots) == NSLOT
    doff = np.array([di * PADDED + dj for di, dj in slots], dtype=np.int32)
    dif = np.array([di for di, dj in slots], dtype=np.float32)
    djf = np.array([dj for di, dj in slots], dtype=np.float32)
    return doff, dif, djf


def _project_body(mat_ref, x_ref, y_ref, z_ref,
                  a_ref, fi_ref, fj_ref, ft_ref):
    x = x_ref[...].astype(jnp.bfloat16).astype(jnp.float32)
    y = y_ref[...].astype(jnp.bfloat16).astype(jnp.float32)
    z = z_ref[...].astype(jnp.bfloat16).astype(jnp.float32)
    m = [mat_ref[i].astype(jnp.bfloat16).astype(jnp.float32)
         for i in range(12)]
    px = ((x * m[0] + y * m[1]) + z * m[2]) + m[3]
    py = ((x * m[4] + y * m[5]) + z * m[6]) + m[7]
    pz = ((x * m[8] + y * m[9]) + z * m[10]) + m[11]
    pi = (-py + 1.0) / 2.0 * float(IMAGE_SIZE - 1)
    pj = (px + 1.0) / 2.0 * float(IMAGE_SIZE - 1)
    cic = jnp.clip(jnp.round(pi), 0.0, float(IMAGE_SIZE - 1))
    cjc = jnp.clip(jnp.round(pj), 0.0, float(IMAGE_SIZE - 1))
    zmin = jnp.min(pz)
    zmax = jnp.max(pz)
    a_ref[...] = (cic * float(PADDED) + cjc).astype(jnp.int32) + (
        PAD * PADDED + PAD)
    fi_ref[...] = pi - cic
    fj_ref[...] = pj - cjc
    ft_ref[...] = 1.0 - (pz - zmin) / (zmax - zmin)


def _merge_body(p_ref, o_ref):
    x = p_ref[:, :, PAD:PAD + IMAGE_SIZE, PAD:PAD + IMAGE_SIZE]
    o_ref[...] = jnp.maximum(jnp.max(x, axis=1, keepdims=True), 0.0)


def _scatter_body(a_hbm, fi_hbm, fj_hbm, ft_hbm,
                  dio_hbm, dif_hbm, djf_hbm, zero_hbm, out_hbm,
                  canvas, a_v, fi_v, fj_v, ft_v, dio_v, dif_v, djf_v):
    wid = lax.axis_index("s") * 2 + lax.axis_index("c")
    base = wid * PTS_PER_SUB
    pltpu.sync_copy(a_hbm.at[pl.ds(base, PTS_PER_SUB)], a_v)
    pltpu.sync_copy(fi_hbm.at[pl.ds(base, PTS_PER_SUB)], fi_v)
    pltpu.sync_copy(fj_hbm.at[pl.ds(base, PTS_PER_SUB)], fj_v)
    pltpu.sync_copy(ft_hbm.at[pl.ds(base, PTS_PER_SUB)], ft_v)
    pltpu.sync_copy(dio_hbm, dio_v)
    pltpu.sync_copy(dif_hbm, dif_v)
    pltpu.sync_copy(djf_hbm, djf_v)
    pltpu.sync_copy(zero_hbm, canvas)

    nk = NSLOT // 16
    dio = [dio_v[pl.ds(k * 16, 16)] for k in range(nk)]
    dif = [dif_v[pl.ds(k * 16, 16)] for k in range(nk)]
    djf = [djf_v[pl.ds(k * 16, 16)] for k in range(nk)]

    def point_av(p):
        idxp = jnp.full((16,), p, dtype=jnp.int32)
        a0 = plsc.load_gather(a_v, [idxp])
        fib = plsc.load_gather(fi_v, [idxp])
        fjb = plsc.load_gather(fj_v, [idxp])
        ftb = plsc.load_gather(ft_v, [idxp])
        cf = [ftb * jnp.float32(c) for c in _POLY]
        addrs, vals = [], []
        for k in range(nk):
            addrs.append(a0 + dio[k])
            dx = dif[k] - fib
            dy = djf[k] - fjb
            d2 = jnp.minimum(dx * dx + dy * dy, 60.0)
            w = cf[4]
            for c in (cf[3], cf[2], cf[1], cf[0]):
                w = w * d2 + c
            vals.append(w)
        return addrs, vals

    def point_rmw(addrs, vals):
        olds = [plsc.load_gather(canvas, [a]) for a in addrs]
        for k in range(nk):
            plsc.store_scatter(canvas, [addrs[k]],
                               jnp.maximum(olds[k], vals[k]))

    def body(p, carry):
        aa, va = point_av(p)
        point_rmw(aa, va)
        return carry

    lax.fori_loop(0, PTS_PER_SUB, body, 0)
    pltpu.sync_copy(canvas, out_hbm.at[wid])


def _project(mat, x2, y2, z2):
    shp = x2.shape
    return pl.pallas_call(
        _project_body,
        in_specs=[pl.BlockSpec(memory_space=pltpu.SMEM),
                  pl.BlockSpec(memory_space=pltpu.VMEM),
                  pl.BlockSpec(memory_space=pltpu.VMEM),
                  pl.BlockSpec(memory_space=pltpu.VMEM)],
        out_specs=[pl.BlockSpec(memory_space=pltpu.VMEM)] * 4,
        out_shape=[jax.ShapeDtypeStruct(shp, jnp.int32),
                   jax.ShapeDtypeStruct(shp, jnp.float32),
                   jax.ShapeDtypeStruct(shp, jnp.float32),
                   jax.ShapeDtypeStruct(shp, jnp.float32)],
    )(mat, x2, y2, z2)


def _merge(partials):
    return pl.pallas_call(
        _merge_body,
        grid=(4,),
        in_specs=[pl.BlockSpec((1, 8, PADDED, PADDED),
                               lambda b: (b, 0, 0, 0))],
        out_specs=pl.BlockSpec((1, 1, IMAGE_SIZE, IMAGE_SIZE),
                               lambda b: (b, 0, 0, 0)),
        out_shape=jax.ShapeDtypeStruct((4, 1, IMAGE_SIZE, IMAGE_SIZE),
                                       jnp.float32),
    )(partials)


def _make_scatter():
  return functools.partial(
    pl.kernel,
    mesh=plsc.VectorSubcoreMesh(core_axis_name="c", subcore_axis_name="s"),
    compiler_params=pltpu.CompilerParams(needs_layout_passes=False),
    out_type=jax.ShapeDtypeStruct((N_SUBCORES, _HWP), jnp.float32),
    scratch_types=[
        pltpu.VMEM((_HWP,), jnp.float32),
        pltpu.VMEM((PTS_PER_SUB,), jnp.int32),
        pltpu.VMEM((PTS_PER_SUB,), jnp.float32),
        pltpu.VMEM((PTS_PER_SUB,), jnp.float32),
        pltpu.VMEM((PTS_PER_SUB,), jnp.float32),
        pltpu.VMEM((NSLOT,), jnp.int32),
        pltpu.VMEM((NSLOT,), jnp.float32),
        pltpu.VMEM((NSLOT,), jnp.float32),
    ],
  )(_scatter_body)


def kernel(data, view_id):
    pre = _pre_matrices()
    mat = pre[view_id]
    pcds = data.reshape(-1, 3)
    x2 = pcds[:, 0].reshape(512, 128)
    y2 = pcds[:, 1].reshape(512, 128)
    z2 = pcds[:, 2].reshape(512, 128)
    matv = mat[:3].reshape(12)

    a2, fi2, fj2, ft2 = _project(matv, x2, y2, z2)

    doff, dif, djf = _offset_tables()
    zero = jnp.zeros((_HWP,), dtype=jnp.float32)
    partials = _make_scatter()(
        a2.reshape(N_POINTS), fi2.reshape(N_POINTS),
        fj2.reshape(N_POINTS), ft2.reshape(N_POINTS),
        jnp.asarray(doff), jnp.asarray(dif), jnp.asarray(djf), zero)

    return _merge(partials.reshape(4, 8, PADDED, PADDED))


# speedup vs baseline: 1.1071x; 1.1071x over previous
import functools
import math

import jax
import jax.numpy as jnp
import numpy as np
from jax import lax
from jax.experimental import pallas as pl
from jax.experimental.pallas import tpu as pltpu
from jax.experimental.pallas import tpu_sc as plsc

IMAGE_SIZE = 256
PAD = 5
PADDED = IMAGE_SIZE + 2 * PAD
_HWP = PADDED * PADDED
N_POINTS = 4 * 16384
N_SUBCORES = 32
PTS_PER_SUB = N_POINTS // N_SUBCORES
KR = 5
NSLOT = 112

_EYES = np.array(
    [[-1, -1, -1], [-1, -1, 1], [-1, 1, -1], [-1, 1, 1],
     [1, -1, -1], [1, -1, 1], [1, 1, -1], [1, 1, 1]], dtype=np.float32)

_C1 = float((np.pi / 10.0) ** 2)
_POLY = [(-1.0) ** k * _C1 ** k / float(math.factorial(2 * k))
         for k in range(5)]


def _normalize(x):
    n = jnp.linalg.norm(x, axis=1, keepdims=True)
    return x / jnp.maximum(n, 1e-6)


def _look_at(eyes, centers, ups):
    B = eyes.shape[0]
    zaxis = _normalize(eyes - centers)
    xaxis = _normalize(jnp.cross(ups, zaxis))
    yaxis = jnp.cross(zaxis, xaxis)
    z = jnp.zeros([B], dtype=eyes.dtype)
    o = jnp.ones([B], dtype=eyes.dtype)
    translation = jnp.stack(
        [o, z, z, -eyes[:, 0], z, o, z, -eyes[:, 1], z, z, o, -eyes[:, 2],
         z, z, z, o], -1).reshape(-1, 4, 4)
    orientation = jnp.stack(
        [xaxis[:, 0], xaxis[:, 1], xaxis[:, 2], z,
         yaxis[:, 0], yaxis[:, 1], yaxis[:, 2], z,
         zaxis[:, 0], zaxis[:, 1], zaxis[:, 2], z, z, z, z, o],
        -1).reshape(-1, 4, 4)
    return orientation @ translation


def _orthorgonal(scalex, scaley, z_near, z_far):
    z = jnp.zeros_like(z_near)
    o = jnp.ones_like(z_near)
    k1 = -2.0 / (z_far - z_near)
    k2 = (z_far + z_near) / (z_far - z_near)
    return jnp.stack(
        [scalex, z, z, z, z, scaley, z, z, z, z, k1, k2, z, z, z, o],
        -1).reshape(-1, 4, 4)


def _pre_matrices():
    proj = _orthorgonal(jnp.array([1.5], dtype=jnp.float32),
                        jnp.array([1.5], dtype=jnp.float32),
                        jnp.array([0.1], dtype=jnp.float32),
                        jnp.array([10.0], dtype=jnp.float32))
    mats = []
    for i in range(_EYES.shape[0]):
        vm = _look_at(jnp.asarray(_EYES[i:i + 1]),
                      jnp.zeros((1, 3), dtype=jnp.float32),
                      jnp.array([[0.0, 0.0, 1.0]], dtype=jnp.float32))
        mats.append(proj @ vm)
    return jnp.concatenate(mats, axis=0)


def _offset_tables():
    live, dead = [], []
    for di in range(-KR, KR + 1):
        for dj in range(-KR, KR + 1):
            m = max(abs(di) - 0.5, 0.0) ** 2 + max(abs(dj) - 0.5, 0.0) ** 2
            (live if m <= 25.0 else dead).append((di, dj))
    slots = live + dead[:NSLOT - len(live)]
    assert len(slots) == NSLOT
    doff = np.array([di * PADDED + dj for di, dj in slots], dtype=np.int32)
    dif = np.array([di for di, dj in slots], dtype=np.float32)
    djf = np.array([dj for di, dj in slots], dtype=np.float32)
    return doff, dif, djf


def _project_body(mat_ref, x_ref, y_ref, z_ref,
                  a_ref, fi_ref, fj_ref, ft_ref):
    x = x_ref[...].astype(jnp.bfloat16).astype(jnp.float32)
    y = y_ref[...].astype(jnp.bfloat16).astype(jnp.float32)
    z = z_ref[...].astype(jnp.bfloat16).astype(jnp.float32)
    m = [mat_ref[i].astype(jnp.bfloat16).astype(jnp.float32)
         for i in range(12)]
    px = ((x * m[0] + y * m[1]) + z * m[2]) + m[3]
    py = ((x * m[4] + y * m[5]) + z * m[6]) + m[7]
    pz = ((x * m[8] + y * m[9]) + z * m[10]) + m[11]
    pi = (-py + 1.0) / 2.0 * float(IMAGE_SIZE - 1)
    pj = (px + 1.0) / 2.0 * float(IMAGE_SIZE - 1)
    cic = jnp.clip(jnp.round(pi), 0.0, float(IMAGE_SIZE - 1))
    cjc = jnp.clip(jnp.round(pj), 0.0, float(IMAGE_SIZE - 1))
    zmin = jnp.min(pz)
    zmax = jnp.max(pz)
    a_ref[...] = (cic * float(PADDED) + cjc).astype(jnp.int32) + (
        PAD * PADDED + PAD)
    fi_ref[...] = pi - cic
    fj_ref[...] = pj - cjc
    ft_ref[...] = 1.0 - (pz - zmin) / (zmax - zmin)


def _merge_body(p_ref, o_ref):
    o_ref[...] = jnp.maximum(jnp.max(p_ref[...], axis=1, keepdims=True), 0.0)


def _scatter_body(a_hbm, fi_hbm, fj_hbm, ft_hbm,
                  dio_hbm, dif_hbm, djf_hbm, zero_hbm, out_hbm,
                  canvas, a_v, fi_v, fj_v, ft_v, dio_v, dif_v, djf_v):
    wid = lax.axis_index("s") * 2 + lax.axis_index("c")
    base = wid * PTS_PER_SUB
    pltpu.sync_copy(a_hbm.at[pl.ds(base, PTS_PER_SUB)], a_v)
    pltpu.sync_copy(fi_hbm.at[pl.ds(base, PTS_PER_SUB)], fi_v)
    pltpu.sync_copy(fj_hbm.at[pl.ds(base, PTS_PER_SUB)], fj_v)
    pltpu.sync_copy(ft_hbm.at[pl.ds(base, PTS_PER_SUB)], ft_v)
    pltpu.sync_copy(dio_hbm, dio_v)
    pltpu.sync_copy(dif_hbm, dif_v)
    pltpu.sync_copy(djf_hbm, djf_v)
    pltpu.sync_copy(zero_hbm, canvas)

    nk = NSLOT // 16
    dio = [dio_v[pl.ds(k * 16, 16)] for k in range(nk)]
    dif = [dif_v[pl.ds(k * 16, 16)] for k in range(nk)]
    djf = [djf_v[pl.ds(k * 16, 16)] for k in range(nk)]

    def point_av(p):
        idxp = jnp.full((16,), p, dtype=jnp.int32)
        a0 = plsc.load_gather(a_v, [idxp])
        fib = plsc.load_gather(fi_v, [idxp])
        fjb = plsc.load_gather(fj_v, [idxp])
        ftb = plsc.load_gather(ft_v, [idxp])
        cf = [ftb * jnp.float32(c) for c in _POLY]
        addrs, vals = [], []
        for k in range(nk):
            addrs.append(a0 + dio[k])
            dx = dif[k] - fib
            dy = djf[k] - fjb
            d2 = jnp.minimum(dx * dx + dy * dy, 60.0)
            w = cf[4]
            for c in (cf[3], cf[2], cf[1], cf[0]):
                w = w * d2 + c
            vals.append(w)
        return addrs, vals

    def point_rmw(addrs, vals):
        olds = [plsc.load_gather(canvas, [a]) for a in addrs]
        for k in range(nk):
            plsc.store_scatter(canvas, [addrs[k]],
                               jnp.maximum(olds[k], vals[k]))

    def body(p, carry):
        p0 = p * 2
        aa, va = point_av(p0)
        ab, vb = point_av(p0 + 1)
        point_rmw(aa, va)
        point_rmw(ab, vb)
        return carry

    lax.fori_loop(0, PTS_PER_SUB // 2, body, 0)
    pltpu.sync_copy(canvas, out_hbm.at[wid])


def _project(mat, x2, y2, z2):
    shp = x2.shape
    return pl.pallas_call(
        _project_body,
        in_specs=[pl.BlockSpec(memory_space=pltpu.SMEM),
                  pl.BlockSpec(memory_space=pltpu.VMEM),
                  pl.BlockSpec(memory_space=pltpu.VMEM),
                  pl.BlockSpec(memory_space=pltpu.VMEM)],
        out_specs=[pl.BlockSpec(memory_space=pltpu.VMEM)] * 4,
        out_shape=[jax.ShapeDtypeStruct(shp, jnp.int32),
                   jax.ShapeDtypeStruct(shp, jnp.float32),
                   jax.ShapeDtypeStruct(shp, jnp.float32),
                   jax.ShapeDtypeStruct(shp, jnp.float32)],
    )(mat, x2, y2, z2)


def _merge(partials):
    return pl.pallas_call(
        _merge_body,
        grid=(4,),
        in_specs=[pl.BlockSpec((1, 8, _HWP), lambda b: (b, 0, 0))],
        out_specs=pl.BlockSpec((1, 1, _HWP), lambda b: (b, 0, 0)),
        out_shape=jax.ShapeDtypeStruct((4, 1, _HWP), jnp.float32),
    )(partials)


def _make_scatter():
  return functools.partial(
    pl.kernel,
    mesh=plsc.VectorSubcoreMesh(core_axis_name="c", subcore_axis_name="s"),
    compiler_params=pltpu.CompilerParams(needs_layout_passes=False),
    out_type=jax.ShapeDtypeStruct((N_SUBCORES, _HWP), jnp.float32),
    scratch_types=[
        pltpu.VMEM((_HWP,), jnp.float32),
        pltpu.VMEM((PTS_PER_SUB,), jnp.int32),
        pltpu.VMEM((PTS_PER_SUB,), jnp.float32),
        pltpu.VMEM((PTS_PER_SUB,), jnp.float32),
        pltpu.VMEM((PTS_PER_SUB,), jnp.float32),
        pltpu.VMEM((NSLOT,), jnp.int32),
        pltpu.VMEM((NSLOT,), jnp.float32),
        pltpu.VMEM((NSLOT,), jnp.float32),
    ],
  )(_scatter_body)


def kernel(data, view_id):
    pre = _pre_matrices()
    mat = pre[view_id]
    pcds = data.reshape(-1, 3)
    x2 = pcds[:, 0].reshape(512, 128)
    y2 = pcds[:, 1].reshape(512, 128)
    z2 = pcds[:, 2].reshape(512, 128)
    matv = mat[:3].reshape(12)

    a2, fi2, fj2, ft2 = _project(matv, x2, y2, z2)

    doff, dif, djf = _offset_tables()
    zero = jnp.zeros((_HWP,), dtype=jnp.float32)
    partials = _make_scatter()(
        a2.reshape(N_POINTS), fi2.reshape(N_POINTS),
        fj2.reshape(N_POINTS), ft2.reshape(N_POINTS),
        jnp.asarray(doff), jnp.asarray(dif), jnp.asarray(djf), zero)

    merged = _merge(partials.reshape(4, 8, _HWP))
    padded = merged.reshape(4, 1, PADDED, PADDED)
    return padded[:, :, PAD:PAD + IMAGE_SIZE, PAD:PAD + IMAGE_SIZE]


# speedup vs baseline: 1.6936x; 1.5298x over previous
import functools
import math

import jax
import jax.numpy as jnp
import numpy as np
from jax import lax
from jax.experimental import pallas as pl
from jax.experimental.pallas import tpu as pltpu
from jax.experimental.pallas import tpu_sc as plsc

IMAGE_SIZE = 256
PAD = 5
PADDED = IMAGE_SIZE + 2 * PAD
_HWP = PADDED * PADDED
N_POINTS = 4 * 16384
N_SUBCORES = 32
PTS_PER_SUB = N_POINTS // N_SUBCORES
KR = 5
NSLOT = 112

_EYES = np.array(
    [[-1, -1, -1], [-1, -1, 1], [-1, 1, -1], [-1, 1, 1],
     [1, -1, -1], [1, -1, 1], [1, 1, -1], [1, 1, 1]], dtype=np.float32)

_C1 = float((np.pi / 10.0) ** 2)
_POLY = [(-1.0) ** k * _C1 ** k / float(math.factorial(2 * k))
         for k in range(5)]


def _normalize(x):
    n = jnp.linalg.norm(x, axis=1, keepdims=True)
    return x / jnp.maximum(n, 1e-6)


def _look_at(eyes, centers, ups):
    B = eyes.shape[0]
    zaxis = _normalize(eyes - centers)
    xaxis = _normalize(jnp.cross(ups, zaxis))
    yaxis = jnp.cross(zaxis, xaxis)
    z = jnp.zeros([B], dtype=eyes.dtype)
    o = jnp.ones([B], dtype=eyes.dtype)
    translation = jnp.stack(
        [o, z, z, -eyes[:, 0], z, o, z, -eyes[:, 1], z, z, o, -eyes[:, 2],
         z, z, z, o], -1).reshape(-1, 4, 4)
    orientation = jnp.stack(
        [xaxis[:, 0], xaxis[:, 1], xaxis[:, 2], z,
         yaxis[:, 0], yaxis[:, 1], yaxis[:, 2], z,
         zaxis[:, 0], zaxis[:, 1], zaxis[:, 2], z, z, z, z, o],
        -1).reshape(-1, 4, 4)
    return orientation @ translation


def _orthorgonal(scalex, scaley, z_near, z_far):
    z = jnp.zeros_like(z_near)
    o = jnp.ones_like(z_near)
    k1 = -2.0 / (z_far - z_near)
    k2 = (z_far + z_near) / (z_far - z_near)
    return jnp.stack(
        [scalex, z, z, z, z, scaley, z, z, z, z, k1, k2, z, z, z, o],
        -1).reshape(-1, 4, 4)


def _pre_matrices():
    proj = _orthorgonal(jnp.array([1.5], dtype=jnp.float32),
                        jnp.array([1.5], dtype=jnp.float32),
                        jnp.array([0.1], dtype=jnp.float32),
                        jnp.array([10.0], dtype=jnp.float32))
    mats = []
    for i in range(_EYES.shape[0]):
        vm = _look_at(jnp.asarray(_EYES[i:i + 1]),
                      jnp.zeros((1, 3), dtype=jnp.float32),
                      jnp.array([[0.0, 0.0, 1.0]], dtype=jnp.float32))
        mats.append(proj @ vm)
    return jnp.concatenate(mats, axis=0)


def _offset_tables():
    live, dead = [], []
    for di in range(-KR, KR + 1):
        for dj in range(-KR, KR + 1):
            m = max(abs(di) - 0.5, 0.0) ** 2 + max(abs(dj) - 0.5, 0.0) ** 2
            (live if m <= 25.0 else dead).append((di, dj))
    slots = live + dead[:NSLOT - len(live)]
    assert len(slots) == NSLOT
    doff = np.array([di * PADDED + dj for di, dj in slots], dtype=np.int32)
    dif = np.array([di for di, dj in slots], dtype=np.float32)
    djf = np.array([dj for di, dj in slots], dtype=np.float32)
    return doff, dif, djf


def _project_body(mat_ref, x_ref, y_ref, z_ref,
                  a_ref, fi_ref, fj_ref, ft_ref):
    x = x_ref[...].astype(jnp.bfloat16).astype(jnp.float32)
    y = y_ref[...].astype(jnp.bfloat16).astype(jnp.float32)
    z = z_ref[...].astype(jnp.bfloat16).astype(jnp.float32)
    m = [mat_ref[i].astype(jnp.bfloat16).astype(jnp.float32)
         for i in range(12)]
    px = ((x * m[0] + y * m[1]) + z * m[2]) + m[3]
    py = ((x * m[4] + y * m[5]) + z * m[6]) + m[7]
    pz = ((x * m[8] + y * m[9]) + z * m[10]) + m[11]
    pi = (-py + 1.0) / 2.0 * float(IMAGE_SIZE - 1)
    pj = (px + 1.0) / 2.0 * float(IMAGE_SIZE - 1)
    cic = jnp.clip(jnp.round(pi), 0.0, float(IMAGE_SIZE - 1))
    cjc = jnp.clip(jnp.round(pj), 0.0, float(IMAGE_SIZE - 1))
    zmin = jnp.min(pz)
    zmax = jnp.max(pz)
    a_ref[...] = (cic * float(PADDED) + cjc).astype(jnp.int32) + (
        PAD * PADDED + PAD)
    fi_ref[...] = pi - cic
    fj_ref[...] = pj - cjc
    ft_ref[...] = 1.0 - (pz - zmin) / (zmax - zmin)


def _merge_body(p_ref, o_ref):
    o_ref[...] = jnp.maximum(jnp.max(p_ref[...], axis=1, keepdims=True), 0.0)


def _scatter_body(a_hbm, fi_hbm, fj_hbm, ft_hbm,
                  dio_hbm, dif_hbm, djf_hbm, zero_hbm, out_hbm,
                  canvas, a_v, fi_v, fj_v, ft_v, dio_v, dif_v, djf_v,
                  ca_v, cfi_v, cfj_v, cft_v):
    wid = lax.axis_index("s") * 2 + lax.axis_index("c")
    base = wid * PTS_PER_SUB
    pltpu.sync_copy(a_hbm.at[pl.ds(base, PTS_PER_SUB)], a_v)
    pltpu.sync_copy(fi_hbm.at[pl.ds(base, PTS_PER_SUB)], fi_v)
    pltpu.sync_copy(fj_hbm.at[pl.ds(base, PTS_PER_SUB)], fj_v)
    pltpu.sync_copy(ft_hbm.at[pl.ds(base, PTS_PER_SUB)], ft_v)
    pltpu.sync_copy(dio_hbm, dio_v)
    pltpu.sync_copy(dif_hbm, dif_v)
    pltpu.sync_copy(djf_hbm, djf_v)
    pltpu.sync_copy(zero_hbm, canvas)

    def cull(i, nkept):
        sl = pl.ds(i * 16, 16)
        fiv = fi_v[sl]
        fjv = fj_v[sl]
        keep = (fiv * fiv <= 25.0) & (fjv * fjv <= 25.0)
        cnt = plsc.all_reduce_population_count(keep)
        dst = pl.ds(nkept, 16)
        plsc.store_compressed(ca_v.at[dst], a_v[sl], mask=keep)
        plsc.store_compressed(cfi_v.at[dst], fiv, mask=keep)
        plsc.store_compressed(cfj_v.at[dst], fjv, mask=keep)
        plsc.store_compressed(cft_v.at[dst], ft_v[sl], mask=keep)
        return nkept + jnp.max(cnt)

    n = lax.fori_loop(0, PTS_PER_SUB // 16, cull, jnp.int32(0))

    lane0 = jax.lax.iota(jnp.int32, 16) == 0
    dmask = lane0 & ((n % 2) == 1)
    didx = jnp.full((16,), jnp.minimum(n, PTS_PER_SUB - 1), dtype=jnp.int32)
    plsc.store_scatter(ca_v, [didx],
                       jnp.full((16,), PAD * PADDED + PAD, dtype=jnp.int32),
                       mask=dmask)
    plsc.store_scatter(cfi_v, [didx], jnp.zeros((16,), jnp.float32),
                       mask=dmask)
    plsc.store_scatter(cfj_v, [didx], jnp.zeros((16,), jnp.float32),
                       mask=dmask)
    plsc.store_scatter(cft_v, [didx], jnp.zeros((16,), jnp.float32),
                       mask=dmask)

    nk = NSLOT // 16
    dio = [dio_v[pl.ds(k * 16, 16)] for k in range(nk)]
    dif = [dif_v[pl.ds(k * 16, 16)] for k in range(nk)]
    djf = [djf_v[pl.ds(k * 16, 16)] for k in range(nk)]

    def point_av(p):
        idxp = jnp.full((16,), p, dtype=jnp.int32)
        a0 = plsc.load_gather(ca_v, [idxp])
        fib = plsc.load_gather(cfi_v, [idxp])
        fjb = plsc.load_gather(cfj_v, [idxp])
        ftb = plsc.load_gather(cft_v, [idxp])
        cf = [ftb * jnp.float32(c) for c in _POLY]
        addrs, vals = [], []
        for k in range(nk):
            addrs.append(a0 + dio[k])
            dx = dif[k] - fib
            dy = djf[k] - fjb
            d2 = jnp.minimum(dx * dx + dy * dy, 60.0)
            w = cf[4]
            for c in (cf[3], cf[2], cf[1], cf[0]):
                w = w * d2 + c
            vals.append(w)
        return addrs, vals

    def point_rmw(addrs, vals):
        olds = [plsc.load_gather(canvas, [a]) for a in addrs]
        for k in range(nk):
            plsc.store_scatter(canvas, [addrs[k]],
                               jnp.maximum(olds[k], vals[k]))

    def body(p, carry):
        p0 = p * 2
        aa, va = point_av(p0)
        ab, vb = point_av(p0 + 1)
        point_rmw(aa, va)
        point_rmw(ab, vb)
        return carry

    lax.fori_loop(0, (n + 1) // 2, body, 0)
    pltpu.sync_copy(canvas, out_hbm.at[wid])


def _project(mat, x2, y2, z2):
    shp = x2.shape
    return pl.pallas_call(
        _project_body,
        in_specs=[pl.BlockSpec(memory_space=pltpu.SMEM),
                  pl.BlockSpec(memory_space=pltpu.VMEM),
                  pl.BlockSpec(memory_space=pltpu.VMEM),
                  pl.BlockSpec(memory_space=pltpu.VMEM)],
        out_specs=[pl.BlockSpec(memory_space=pltpu.VMEM)] * 4,
        out_shape=[jax.ShapeDtypeStruct(shp, jnp.int32),
                   jax.ShapeDtypeStruct(shp, jnp.float32),
                   jax.ShapeDtypeStruct(shp, jnp.float32),
                   jax.ShapeDtypeStruct(shp, jnp.float32)],
    )(mat, x2, y2, z2)


def _merge(partials):
    return pl.pallas_call(
        _merge_body,
        grid=(4,),
        in_specs=[pl.BlockSpec((1, 8, _HWP), lambda b: (b, 0, 0))],
        out_specs=pl.BlockSpec((1, 1, _HWP), lambda b: (b, 0, 0)),
        out_shape=jax.ShapeDtypeStruct((4, 1, _HWP), jnp.float32),
    )(partials)


def _make_scatter():
  return functools.partial(
    pl.kernel,
    mesh=plsc.VectorSubcoreMesh(core_axis_name="c", subcore_axis_name="s"),
    compiler_params=pltpu.CompilerParams(needs_layout_passes=False),
    out_type=jax.ShapeDtypeStruct((N_SUBCORES, _HWP), jnp.float32),
    scratch_types=[
        pltpu.VMEM((_HWP,), jnp.float32),
        pltpu.VMEM((PTS_PER_SUB,), jnp.int32),
        pltpu.VMEM((PTS_PER_SUB,), jnp.float32),
        pltpu.VMEM((PTS_PER_SUB,), jnp.float32),
        pltpu.VMEM((PTS_PER_SUB,), jnp.float32),
        pltpu.VMEM((NSLOT,), jnp.int32),
        pltpu.VMEM((NSLOT,), jnp.float32),
        pltpu.VMEM((NSLOT,), jnp.float32),
        pltpu.VMEM((PTS_PER_SUB,), jnp.int32),
        pltpu.VMEM((PTS_PER_SUB,), jnp.float32),
        pltpu.VMEM((PTS_PER_SUB,), jnp.float32),
        pltpu.VMEM((PTS_PER_SUB,), jnp.float32),
    ],
  )(_scatter_body)


def kernel(data, view_id):
    pre = _pre_matrices()
    mat = pre[view_id]
    pcds = data.reshape(-1, 3)
    x2 = pcds[:, 0].reshape(512, 128)
    y2 = pcds[:, 1].reshape(512, 128)
    z2 = pcds[:, 2].reshape(512, 128)
    matv = mat[:3].reshape(12)

    a2, fi2, fj2, ft2 = _project(matv, x2, y2, z2)

    doff, dif, djf = _offset_tables()
    zero = jnp.zeros((_HWP,), dtype=jnp.float32)
    partials = _make_scatter()(
        a2.reshape(N_POINTS), fi2.reshape(N_POINTS),
        fj2.reshape(N_POINTS), ft2.reshape(N_POINTS),
        jnp.asarray(doff), jnp.asarray(dif), jnp.asarray(djf), zero)

    merged = _merge(partials.reshape(4, 8, _HWP))
    padded = merged.reshape(4, 1, PADDED, PADDED)
    return padded[:, :, PAD:PAD + IMAGE_SIZE, PAD:PAD + IMAGE_SIZE]
